# speedup vs baseline: 1.0089x; 1.0089x over previous
import functools

import jax
import jax.numpy as jnp
from jax.experimental import pallas as pl
from jax.experimental.pallas import tpu as pltpu

N = 50000
E = 800000
F_IN = 86
H = 128
R = 4
NB = 30
G = 32
C = 18

_BN = 1000


def _head_body(h_ref, batch_ref, fcw_ref, fcb_ref, out_ref, acc_ref):
    i = pl.program_id(0)
    nsteps = pl.num_programs(0)

    @pl.when(i == 0)
    def _():
        acc_ref[...] = jnp.zeros_like(acc_ref)

    b = batch_ref[0, 0, :]
    onehot = (jax.lax.broadcasted_iota(jnp.int32, (G, _BN), 0) == b[None, :]).astype(jnp.float32)
    acc_ref[...] += jax.lax.dot_general(
        onehot, h_ref[...], (((1,), (0,)), ((), ())), preferred_element_type=jnp.float32
    )

    @pl.when(i == nsteps - 1)
    def _():
        g = acc_ref[...]
        z = jax.nn.relu(
            jax.lax.dot_general(g, fcw_ref[...], (((1,), (0,)), ((), ())),
                                preferred_element_type=jnp.float32)
            + fcb_ref[...][None, :]
        )
        m = jnp.max(z, axis=1, keepdims=True)
        s = z - m
        lse = jnp.log(jnp.sum(jnp.exp(s), axis=1, keepdims=True))
        out_ref[...] = s - lse


def _head_pallas(h, batch, fc_w, fc_b):
    batch3 = batch.reshape(N // _BN, 1, _BN)
    grid = (N // _BN,)
    return pl.pallas_call(
        _head_body,
        grid=grid,
        in_specs=[
            pl.BlockSpec((_BN, H), lambda i: (i, 0)),
            pl.BlockSpec((1, 1, _BN), lambda i: (i, 0, 0)),
            pl.BlockSpec((H, C), lambda i: (0, 0)),
            pl.BlockSpec((C,), lambda i: (0,)),
        ],
        out_specs=pl.BlockSpec((G, C), lambda i: (0, 0)),
        out_shape=jax.ShapeDtypeStruct((G, C), jnp.float32),
        scratch_shapes=[pltpu.VMEM((G, H), jnp.float32)],
    )(h, batch3, fc_w, fc_b)


def _rgcn_conv(x, edge_index, edge_type, comp, bases, root, bias):
    nb, in_c, out_c = bases.shape
    num_rel = comp.shape[0]
    weight = (comp @ bases.reshape(nb, in_c * out_c)).reshape(num_rel, in_c, out_c)
    src = edge_index[0]
    dst = edge_index[1]
    n = x.shape[0]
    out = x @ root + bias
    for r in range(num_rel):
        mask = (edge_type == r).astype(x.dtype)
        msg = (x @ weight[r])[src] * mask[:, None]
        s = jax.ops.segment_sum(msg, dst, num_segments=n)
        cnt = jax.ops.segment_sum(mask, dst, num_segments=n)
        out = out + s / jnp.maximum(cnt, 1.0)[:, None]
    return out


def kernel(x, edge_index, edge_type, batch, bases1, comp1, root1, bias1,
           bases2, comp2, root2, bias2, fc_w, fc_b):
    h = jax.nn.relu(_rgcn_conv(x, edge_index, edge_type, comp1, bases1, root1, bias1))
    h = _rgcn_conv(h, edge_index, edge_type, comp2, bases2, root2, bias2)
    return _head_pallas(h, batch, fc_w, fc_b)


# speedup vs baseline: 7.2113x; 7.1475x over previous
import functools

import jax
import jax.numpy as jnp
from jax import lax
from jax.experimental import pallas as pl
from jax.experimental.pallas import tpu as pltpu
from jax.experimental.pallas import tpu_sc as plsc

N = 50000
E = 800000
F_IN = 86
H = 128
R = 4
NB = 30
G = 32
C = 18

BN = 1024
NBLK = 49
NP = BN * NBLK
HC = 32
BH = 1000

NSC = 2
NTILE = 16
SG = 128
KE = 1024
E_PAD = NSC * NTILE * 8 * SG * 25
ROWS = E_PAD // SG
RPT_ALL = ROWS // (NSC * NTILE)
CPT_ALL = RPT_ALL // 8
RPT_SC = ROWS // NTILE
CPT_SC = RPT_SC // 8
NPS = NP // NTILE
NSTG = 49
NSTG_IT = NPS // NSTG
ACC_ROWS = NP + 16
CNT_SZ = 200960
CSLC = CNT_SZ // NTILE

_sc_mesh = plsc.VectorSubcoreMesh(core_axis_name="c", subcore_axis_name="s")


def _prep_body(src_hbm, dst_hbm, typ_hbm,
               gidx_hbm, scal_hbm,
               cnt_sp, sv, dv, tv, gv, cv, sb, ones_v, z_v):
    cid = lax.axis_index("c")
    sid = lax.axis_index("s")
    tid = cid * NTILE + sid
    base = sid * CSLC

    def zfill(k, carry):
        z_v[pl.ds(k * 16, 16)] = jnp.zeros((16,), jnp.float32)
        return carry

    lax.fori_loop(0, CSLC // 16, zfill, 0)
    pltpu.sync_copy(z_v, cnt_sp.at[pl.ds(base, CSLC)])
    for k in range(SG // 16):
        ones_v[pl.ds(k * 16, 16)] = jnp.ones((16,), jnp.float32)
    plsc.subcore_barrier()

    rb0 = sid * RPT_SC

    def cchunk(ci, carry):
        rb = rb0 + ci * 8
        pltpu.sync_copy(src_hbm.at[pl.ds(rb, 8)], sv)
        pltpu.sync_copy(dst_hbm.at[pl.ds(rb, 8)], dv)
        pltpu.sync_copy(typ_hbm.at[pl.ds(rb, 8)], tv)
        for g in range(8):
            for k in range(8):
                sl = pl.ds(k * 16, 16)
                gv[g, sl] = (tv[g, sl] + 1) * NP + sv[g, sl]
                cv[g, sl] = dv[g, sl] * R + tv[g, sl]

        @pl.when(cid == 0)
        def _():
            pltpu.sync_copy(gv, gidx_hbm.at[pl.ds(rb, 8)])

        for g in range(8):
            pltpu.sync_copy(ones_v, cnt_sp.at[cv.at[g]], add=True)
        return carry

    lax.fori_loop(0, CPT_SC, cchunk, 0)
    plsc.subcore_barrier()

    pltpu.sync_copy(cnt_sp.at[pl.ds(base, CSLC)], z_v)

    def inv_chunk(k, carry):
        sl = pl.ds(k * 16, 16)
        iv = 1.0 / jnp.maximum(z_v[sl], 1.0)
        gi = base + k * 16 + lax.iota(jnp.int32, 16)
        z_v[sl] = jnp.where(gi < R * N, iv, 0.0)
        return carry

    lax.fori_loop(0, CSLC // 16, inv_chunk, 0)
    pltpu.sync_copy(z_v, cnt_sp.at[pl.ds(base, CSLC)])
    plsc.subcore_barrier()

    rb1 = tid * RPT_ALL

    def schunk(ci, carry):
        rb = rb1 + ci * 8
        pltpu.sync_copy(dst_hbm.at[pl.ds(rb, 8)], dv)
        pltpu.sync_copy(typ_hbm.at[pl.ds(rb, 8)], tv)
        for g in range(8):
            for k in range(8):
                sl = pl.ds(k * 16, 16)
                cv[g, sl] = dv[g, sl] * R + tv[g, sl]
        for g in range(8):
            pltpu.sync_copy(cnt_sp.at[cv.at[g]], sb.at[g])
        pltpu.sync_copy(sb, scal_hbm.at[pl.ds(rb, 8)])
        return carry

    lax.fori_loop(0, CPT_ALL, schunk, 0)


_prep_call = pl.kernel(
    _prep_body,
    out_type=[jax.ShapeDtypeStruct((ROWS, SG), jnp.int32),
              jax.ShapeDtypeStruct((ROWS, SG), jnp.float32)],
    mesh=_sc_mesh,
    compiler_params=pltpu.CompilerParams(use_tc_tiling_on_sc=False),
    scratch_types=[
        pltpu.VMEM_SHARED((CNT_SZ,), jnp.float32),
        pltpu.VMEM((8, SG), jnp.int32),
        pltpu.VMEM((8, SG), jnp.int32),
        pltpu.VMEM((8, SG), jnp.int32),
        pltpu.VMEM((8, SG), jnp.int32),
        pltpu.VMEM((8, SG), jnp.int32),
        pltpu.VMEM((8, SG), jnp.float32),
        pltpu.VMEM((SG,), jnp.float32),
        pltpu.VMEM((CSLC,), jnp.float32),
    ],
)


def _edge_body(ba, bb, gidx_hbm, dst_hbm, scal_hbm,
               oa, ob,
               acc_sp, gv0, gv1, dv0, dv1, sv0, sv1, rq0, rq1, stg0, stg1,
               gsem0, gsem1, ssem0, ssem1, isem0, isem1):
    cid = lax.axis_index("c")
    sid = lax.axis_index("s")
    gvs = (gv0, gv1)
    dvs = (dv0, dv1)
    svs = (sv0, sv1)
    rqs = (rq0, rq1)
    gsems = (gsem0, gsem1)
    ssems = (ssem0, ssem1)
    isems = (isem0, isem1)

    def fire_idx(rb, b):
        pltpu.async_copy(gidx_hbm.at[pl.ds(rb, 8)], gvs[b], isems[b])
        pltpu.async_copy(dst_hbm.at[pl.ds(rb, 8)], dvs[b], isems[b])
        pltpu.async_copy(scal_hbm.at[pl.ds(rb * SG, KE)], svs[b], isems[b])

    def wait_idx(b):
        pltpu.make_async_copy(gidx_hbm.at[pl.ds(0, 8)], gvs[b], isems[b]).wait()
        pltpu.make_async_copy(dst_hbm.at[pl.ds(0, 8)], dvs[b], isems[b]).wait()
        pltpu.make_async_copy(scal_hbm.at[pl.ds(0, KE)], svs[b], isems[b]).wait()

    def do_chunk(bt, ot):
        stgs = (stg0, stg1)

        def psl(p):
            return pl.ds(sid * NPS + p * NSTG, NSTG)

        pltpu.async_copy(bt.at[psl(0)], stg0, isem0)

        def init2(p2, carry):
            p = p2 * 2
            pltpu.make_async_copy(bt.at[pl.ds(0, NSTG)], stg0, isem0).wait()
            pltpu.async_copy(bt.at[psl(p + 1)], stg1, isem1)
            pltpu.sync_copy(stg0, acc_sp.at[psl(p)])
            pltpu.make_async_copy(bt.at[pl.ds(0, NSTG)], stg1, isem1).wait()

            @pl.when(p + 2 < NSTG_IT)
            def _():
                pltpu.async_copy(bt.at[psl(p + 2)], stg0, isem0)

            pltpu.sync_copy(stg1, acc_sp.at[psl(p + 1)])
            return carry

        lax.fori_loop(0, NSTG_IT // 2, init2, 0)
        plsc.subcore_barrier()
        rbase0 = sid * RPT_SC

        def proc(gv, dv, sv):
            def fire_gather(q, b):
                return [pltpu.async_copy(bt.at[gv.at[2 * q + g]],
                                         rqs[b].at[pl.ds(g * SG, SG)], gsems[b])
                        for g in range(2)]

            cps = fire_gather(0, 0)
            scat = [None, None]
            for q in range(4):
                b = q % 2
                nb = 1 - b
                nxt = None
                if q < 3:
                    if scat[nb] is not None:
                        for cp in scat[nb]:
                            cp.wait()
                        scat[nb] = None
                    nxt = fire_gather(q + 1, nb)
                for cp in cps:
                    cp.wait()
                rows = rqs[b]

                @plsc.parallel_loop(0, KE // 4, step=16, unroll=2)
                def _(eb):
                    s16 = sv[pl.ds(q * (KE // 4) + eb, 16)]
                    for u in range(16):
                        s = s16[u]
                        rows[eb + u, pl.ds(0, 16)] = rows[eb + u, pl.ds(0, 16)] * s
                        rows[eb + u, pl.ds(16, 16)] = rows[eb + u, pl.ds(16, 16)] * s

                scat[b] = [pltpu.async_copy(rows.at[pl.ds(g * SG, SG)],
                                            acc_sp.at[dv.at[2 * q + g]],
                                            ssems[b], add=True)
                           for g in range(2)]
                cps = nxt
            for pair in scat:
                if pair is not None:
                    for cp in pair:
                        cp.wait()

        fire_idx(rbase0, 0)

        def two_chunks(ci2, carry):
            ci = ci2 * 2
            wait_idx(0)
            fire_idx(rbase0 + (ci + 1) * 8, 1)
            proc(gv0, dv0, sv0)
            wait_idx(1)

            @pl.when(ci + 2 < CPT_SC)
            def _():
                fire_idx(rbase0 + (ci + 2) * 8, 0)

            proc(gv1, dv1, sv1)
            return carry

        lax.fori_loop(0, CPT_SC // 2, two_chunks, 0)
        plsc.subcore_barrier()

        def out2(p2, carry):
            p = p2 * 2

            @pl.when(p2 > 0)
            def _():
                pltpu.make_async_copy(stg0, ot.at[pl.ds(0, NSTG)], isem0).wait()

            pltpu.sync_copy(acc_sp.at[psl(p)], stg0)
            pltpu.async_copy(stg0, ot.at[psl(p)], isem0)

            @pl.when(p2 > 0)
            def _():
                pltpu.make_async_copy(stg1, ot.at[pl.ds(0, NSTG)], isem1).wait()

            pltpu.sync_copy(acc_sp.at[psl(p + 1)], stg1)
            pltpu.async_copy(stg1, ot.at[psl(p + 1)], isem1)
            return carry

        lax.fori_loop(0, NSTG_IT // 2, out2, 0)
        pltpu.make_async_copy(stg0, ot.at[pl.ds(0, NSTG)], isem0).wait()
        pltpu.make_async_copy(stg1, ot.at[pl.ds(0, NSTG)], isem1).wait()
        plsc.subcore_barrier()

    @pl.when(cid == 0)
    def _():
        do_chunk(ba, oa)

    @pl.when(cid == 1)
    def _():
        do_chunk(bb, ob)


_edge_call = pl.kernel(
    _edge_body,
    out_type=[jax.ShapeDtypeStruct((NP, HC), jnp.float32) for _ in range(2)],
    mesh=_sc_mesh,
    compiler_params=pltpu.CompilerParams(use_tc_tiling_on_sc=False),
    scratch_types=[
        pltpu.VMEM_SHARED((ACC_ROWS, HC), jnp.float32),
        pltpu.VMEM((8, SG), jnp.int32),
        pltpu.VMEM((8, SG), jnp.int32),
        pltpu.VMEM((8, SG), jnp.int32),
        pltpu.VMEM((8, SG), jnp.int32),
        pltpu.VMEM((KE,), jnp.float32),
        pltpu.VMEM((KE,), jnp.float32),
        pltpu.VMEM((KE // 4, HC), jnp.float32),
        pltpu.VMEM((KE // 4, HC), jnp.float32),
        pltpu.VMEM((NSTG, HC), jnp.float32),
        pltpu.VMEM((NSTG, HC), jnp.float32),
        pltpu.SemaphoreType.DMA,
        pltpu.SemaphoreType.DMA,
        pltpu.SemaphoreType.DMA,
        pltpu.SemaphoreType.DMA,
        pltpu.SemaphoreType.DMA,
        pltpu.SemaphoreType.DMA,
    ],
)


def _mm5_body(nx, relu_in, *refs):
    xrefs = refs[:nx]
    w_ref, b_ref = refs[nx], refs[nx + 1]
    outs = refs[nx + 2:]
    r = pl.program_id(1)
    if nx == 1:
        xb = xrefs[0][...]
    else:
        xb = jnp.concatenate([x[...] for x in xrefs], axis=1)
    if relu_in:
        xb = jnp.maximum(xb, 0.0)
    y = jax.lax.dot_general(xb, w_ref[0], (((1,), (0,)), ((), ())),
                            preferred_element_type=jnp.float32)
    y = y + jnp.where(r == 0, 1.0, 0.0) * b_ref[...][None, :]
    for c in range(4):
        outs[c][...] = y[:, c * HC:(c + 1) * HC]


def _mm5(xs, w5, b, relu_in, fdim):
    nx = len(xs)
    xw = HC if nx == 4 else fdim
    body = functools.partial(_mm5_body, nx, relu_in)
    return pl.pallas_call(
        body,
        grid=(NBLK, R + 1),
        in_specs=[pl.BlockSpec((BN, xw), lambda i, r: (i, 0)) for _ in range(nx)]
        + [pl.BlockSpec((1, fdim, H), lambda i, r: (r, 0, 0)),
           pl.BlockSpec((H,), lambda i, r: (0,))],
        out_specs=[pl.BlockSpec((BN, HC), lambda i, r: (r * NBLK + i, 0))
                   for _ in range(4)],
        out_shape=[jax.ShapeDtypeStruct(((R + 1) * NP, HC), jnp.float32)
                   for _ in range(4)],
    )(*xs, w5, b)


def _head_body(h0, h1, h2, h3, batch_ref, fcw_ref, fcb_ref, out_ref, acc_ref):
    i = pl.program_id(0)
    nsteps = pl.num_programs(0)

    @pl.when(i == 0)
    def _():
        acc_ref[...] = jnp.zeros_like(acc_ref)

    b = batch_ref[0, 0, :]
    hb = jnp.concatenate([h0[...], h1[...], h2[...], h3[...]], axis=1)
    onehot = (jax.lax.broadcasted_iota(jnp.int32, (G, BH), 0) == b[None, :]).astype(jnp.float32)
    acc_ref[...] += jax.lax.dot_general(
        onehot, hb, (((1,), (0,)), ((), ())), preferred_element_type=jnp.float32)

    @pl.when(i == nsteps - 1)
    def _():
        gacc = acc_ref[...]
        z = jnp.maximum(
            jax.lax.dot_general(gacc, fcw_ref[...], (((1,), (0,)), ((), ())),
                                preferred_element_type=jnp.float32)
            + fcb_ref[...][None, :], 0.0)
        m = jnp.max(z, axis=1, keepdims=True)
        s = z - m
        lse = jnp.log(jnp.sum(jnp.exp(s), axis=1, keepdims=True))
        out_ref[...] = s - lse


def _head(hs, batch, fc_w, fc_b):
    batch3 = batch.reshape(N // BH, 1, BH)
    return pl.pallas_call(
        _head_body,
        grid=(N // BH,),
        in_specs=[pl.BlockSpec((BH, HC), lambda i: (i, 0)) for _ in range(4)]
        + [pl.BlockSpec((1, 1, BH), lambda i: (i, 0, 0)),
           pl.BlockSpec((H, C), lambda i: (0, 0)),
           pl.BlockSpec((C,), lambda i: (0,))],
        out_specs=pl.BlockSpec((G, C), lambda i: (0, 0)),
        out_shape=jax.ShapeDtypeStruct((G, C), jnp.float32),
        scratch_shapes=[pltpu.VMEM((G, H), jnp.float32)],
    )(*hs, batch3, fc_w, fc_b)


def kernel(x, edge_index, edge_type, batch, bases1, comp1, root1, bias1,
           bases2, comp2, root2, bias2, fc_w, fc_b):
    w1 = (comp1 @ bases1.reshape(NB, F_IN * H)).reshape(R, F_IN, H)
    w5a = jnp.concatenate([root1[None], w1], axis=0)
    w2 = (comp2 @ bases2.reshape(NB, H * H)).reshape(R, H, H)
    w5b = jnp.concatenate([root2[None], w2], axis=0)

    npad = E_PAD - E
    src_p = jnp.concatenate([edge_index[0], jnp.zeros((npad,), jnp.int32)]).reshape(ROWS, SG)
    dst_p = jnp.concatenate([edge_index[1], jnp.full((npad,), NP, jnp.int32)]).reshape(ROWS, SG)
    typ_p = jnp.concatenate([edge_type, jnp.zeros((npad,), jnp.int32)]).reshape(ROWS, SG)

    gidx2d, scal2d = _prep_call(src_p, dst_p, typ_p)
    scal_flat = scal2d.reshape(E_PAD)

    b1t = _mm5([x], w5a, bias1, relu_in=False, fdim=F_IN)
    h1a = _edge_call(b1t[0], b1t[2], gidx2d, dst_p, scal_flat)
    h1b = _edge_call(b1t[1], b1t[3], gidx2d, dst_p, scal_flat)
    h1 = [h1a[0], h1b[0], h1a[1], h1b[1]]

    b2t = _mm5(h1, w5b, bias2, relu_in=True, fdim=H)
    h2a = _edge_call(b2t[0], b2t[2], gidx2d, dst_p, scal_flat)
    h2b = _edge_call(b2t[1], b2t[3], gidx2d, dst_p, scal_flat)
    h2 = [h2a[0], h2b[0], h2a[1], h2b[1]]

    return _head(h2, batch, fc_w, fc_b)
